# Initial kernel scaffold; baseline (speedup 1.0000x reference)
import jax
import jax.numpy as jnp
from jax.experimental import pallas as pl


def kernel(x, edge_index, edge_attr, W_self, W_nbr, W_edge, b, lin_W, lin_b):
    raise NotImplementedError("write your pallas kernel here")


# speedup vs baseline: 2.6614x; 2.6614x over previous
import jax
import jax.numpy as jnp
from jax import lax
from jax.experimental import pallas as pl
from jax.experimental.pallas import tpu as pltpu
from jax.experimental.pallas import tpu_sc as plsc

N = 10000
D = 128
E = 320000
DE = 16
L = 3
C = 2

NT = 32
CHUNK = 128
EPT = 10240
NCHUNKS = EPT // CHUNK
EPAD = NT * EPT
NPAD = 10240
RPT = NPAD // 16
CP = 128

def _segsum_body(table, src3, dst3, zrows, out, acc, sidx, didx, rows, sem):
    c = lax.axis_index("c")
    s = lax.axis_index("s")
    w = c * 16 + s
    pltpu.sync_copy(zrows, acc.at[pl.ds(s * RPT, RPT)])
    pltpu.sync_copy(src3.at[w], sidx)
    pltpu.sync_copy(dst3.at[w], didx)
    plsc.subcore_barrier()

    def chunk(i, carry):
        pltpu.async_copy(table.at[sidx.at[i]], rows, sem).wait()
        pltpu.sync_copy(rows, acc.at[didx.at[i]], add=True)
        return carry

    lax.fori_loop(0, NCHUNKS, chunk, 0)
    plsc.subcore_barrier()
    pltpu.sync_copy(acc.at[pl.ds(s * RPT, RPT)],
                    out.at[pl.ds(c * NPAD + s * RPT, RPT)])


import functools


@functools.lru_cache(maxsize=None)
def _get_segsum():
    return pl.kernel(
        _segsum_body,
        out_type=jax.ShapeDtypeStruct((2 * NPAD, D), jnp.float32),
        mesh=plsc.VectorSubcoreMesh(core_axis_name="c", subcore_axis_name="s"),
        scratch_types=[
            pltpu.VMEM_SHARED((NPAD, D), jnp.float32),
            pltpu.VMEM((NCHUNKS, CHUNK), jnp.int32),
            pltpu.VMEM((NCHUNKS, CHUNK), jnp.int32),
            pltpu.VMEM((CHUNK, D), jnp.float32),
            pltpu.SemaphoreType.DMA,
        ],
    )


BR = 1024
GRID = NPAD // BR


def _tc_layer_body(h_ref, g0_ref, g1_ref, ea0_ref, ea1_ref, ws_ref, wn_ref,
                   we_ref, b_ref, linh_ref, linn_ref, lp_ref,
                   hout_ref, lout_ref):
    h = h_ref[...]
    g = g0_ref[...] + g1_ref[...]
    ea = ea0_ref[...] + ea1_ref[...]
    z = (jnp.dot(h, ws_ref[...], preferred_element_type=jnp.float32, precision=lax.Precision.HIGHEST)
         + jnp.dot(g, wn_ref[...], preferred_element_type=jnp.float32, precision=lax.Precision.HIGHEST)
         + jnp.dot(ea, we_ref[...], preferred_element_type=jnp.float32, precision=lax.Precision.HIGHEST)
         + b_ref[...])
    hn = jnp.maximum(z, 0.0)
    hout_ref[...] = hn
    lout_ref[...] = (lp_ref[...]
                     + jnp.dot(h, linh_ref[...], preferred_element_type=jnp.float32, precision=lax.Precision.HIGHEST)
                     + jnp.dot(hn, linn_ref[...], preferred_element_type=jnp.float32, precision=lax.Precision.HIGHEST))


def _tc_layer(h, gpart, eapart, ws, wn, we, bias, linh, linn, lp):
    return pl.pallas_call(
        _tc_layer_body,
        grid=(GRID,),
        in_specs=[
            pl.BlockSpec((BR, D), lambda i: (i, 0)),
            pl.BlockSpec((BR, D), lambda i: (i, 0)),
            pl.BlockSpec((BR, D), lambda i: (i + GRID, 0)),
            pl.BlockSpec((BR, D), lambda i: (i, 0)),
            pl.BlockSpec((BR, D), lambda i: (i + GRID, 0)),
            pl.BlockSpec((D, D), lambda i: (0, 0)),
            pl.BlockSpec((D, D), lambda i: (0, 0)),
            pl.BlockSpec((D, D), lambda i: (0, 0)),
            pl.BlockSpec((1, D), lambda i: (0, 0)),
            pl.BlockSpec((D, CP), lambda i: (0, 0)),
            pl.BlockSpec((D, CP), lambda i: (0, 0)),
            pl.BlockSpec((BR, CP), lambda i: (i, 0)),
        ],
        out_specs=[
            pl.BlockSpec((BR, D), lambda i: (i, 0)),
            pl.BlockSpec((BR, CP), lambda i: (i, 0)),
        ],
        out_shape=[
            jax.ShapeDtypeStruct((NPAD, D), jnp.float32),
            jax.ShapeDtypeStruct((NPAD, CP), jnp.float32),
        ],
    )(h, gpart, gpart, eapart, eapart, ws, wn, we, bias, linh, linn, lp)


def kernel(x, edge_index, edge_attr, W_self, W_nbr, W_edge, b, lin_W, lin_b):
    src = edge_index[0]
    dst = edge_index[1]
    pad = EPAD - E
    src3 = jnp.concatenate(
        [src, jnp.zeros((pad,), jnp.int32)]).reshape(NT, NCHUNKS, CHUNK)
    dst3 = jnp.concatenate(
        [dst, jnp.full((pad,), N, jnp.int32)]).reshape(NT, NCHUNKS, CHUNK)
    ea128 = jnp.pad(edge_attr, ((0, pad), (0, D - DE)))
    eidx3 = jnp.arange(EPAD, dtype=jnp.int32).reshape(NT, NCHUNKS, CHUNK)
    W_edge_p = jnp.pad(W_edge, ((0, 0), (0, D - DE), (0, 0)))
    xp = jnp.concatenate([x, jnp.zeros((NPAD - N, D), jnp.float32)])
    zrowsD = jnp.zeros((RPT, D), jnp.float32)
    linWp = jnp.pad(lin_W, ((0, 0), (0, 0), (0, CP - C)))
    zeroW = jnp.zeros((D, CP), jnp.float32)
    lb_sum = jnp.pad(jnp.sum(lin_b, axis=0), (0, CP - C))
    logits = jnp.broadcast_to(lb_sum, (NPAD, CP))

    eapart = _get_segsum()(ea128, eidx3, dst3, zrowsD)
    h = xp
    for l in range(L):
        gpart = _get_segsum()(h, src3, dst3, zrowsD)
        linh = linWp[0] if l == 0 else zeroW
        h, logits = _tc_layer(h, gpart, eapart, W_self[l], W_nbr[l],
                              W_edge_p[l], b[l].reshape(1, D), linh,
                              linWp[l + 1], logits)
    return logits[:N, :C]
